# baseline (speedup 1.0000x reference)
import functools

import jax
import jax.numpy as jnp
from jax import lax
from jax.experimental import pallas as pl
from jax.experimental.pallas import tpu as pltpu
from jax.experimental.pallas import tpu_sc as plsc

N_SUBSETS = 3
BATCH = 768
DIM = 2048
N_CLASSES = 16
NBUCKETS = N_SUBSETS * N_CLASSES
LANES = 16
NW = 32
NQ = 16
NH = 2
DSLICE = DIM // NQ
HROWS = BATCH // NH
SUBSET_SHIFT = 8
GROUPS = HROWS // LANES


def _norm_body(p_ref, z_ref, wp_ref, wz_ref, d_ref):
    p = p_ref[...]
    z = z_ref[...]
    pp = jnp.sum(p * p, axis=1, keepdims=True)
    zz = jnp.sum(z * z, axis=1, keepdims=True)
    pz = jnp.sum(p * z, axis=1, keepdims=True)
    wp = 1.0 / jnp.maximum(jnp.sqrt(pp), 1e-8)
    wz = 1.0 / jnp.maximum(jnp.sqrt(zz), 1e-8)
    wp_ref[...] = wp
    wz_ref[...] = wz
    d_ref[...] = wp * wz * pz


def _seg_body(ps_hbm, zs_hbm, tgt_hbm, wp_hbm, wz_hbm,
              outa_hbm, outb_hbm, pcol, zcol, tbuf, wpbuf, wzbuf,
              *accs):
    cid = lax.axis_index("c")
    sid = lax.axis_index("s")
    wid = sid * 2 + cid
    h = wid // NQ
    q = wid % NQ
    rbase = h * HROWS
    nch = DSLICE // LANES

    pltpu.sync_copy(tgt_hbm.at[pl.ds(rbase, HROWS)], tbuf)
    pltpu.sync_copy(wp_hbm.at[pl.ds(rbase, HROWS)], wpbuf)
    pltpu.sync_copy(wz_hbm.at[pl.ds(rbase, HROWS)], wzbuf)
    pltpu.sync_copy(
        ps_hbm.at[pl.ds(rbase, HROWS), pl.ds(q * DSLICE, DSLICE)], pcol)
    pltpu.sync_copy(
        zs_hbm.at[pl.ds(rbase, HROWS), pl.ds(q * DSLICE, DSLICE)], zcol)

    zv = jnp.zeros((LANES,), jnp.float32)

    def _zero(i, c):
        for a in accs:
            a[pl.ds(i * LANES, LANES)] = zv
        return c

    lax.fori_loop(0, NBUCKETS, _zero, 0)

    iota = lax.iota(jnp.int32, LANES)
    gdn = lax.GatherDimensionNumbers(
        offset_dims=(), collapsed_slice_dims=(0,), start_index_map=(0,))

    def _group(g, _):
        gbase = g * LANES
        tv = tbuf[pl.ds(gbase, LANES)]
        wpv = wpbuf[pl.ds(gbase, LANES)]
        wzv = wzbuf[pl.ds(gbase, LANES)]
        rows = iota + jnp.full((LANES,), rbase + gbase, jnp.int32)
        offv = (lax.shift_right_arithmetic(rows, SUBSET_SHIFT) * N_CLASSES
                + tv) * LANES
        for k in range(LANES):
            kfull = jnp.full((LANES, 1), k, jnp.int32)
            bok = lax.gather(offv, kfull, gdn, (1,),
                             mode=lax.GatherScatterMode.PROMISE_IN_BOUNDS)
            idx = bok + iota
            wpk = jnp.full((LANES,), wpv[k], jnp.float32)
            wzk = jnp.full((LANES,), wzv[k], jnp.float32)
            row = gbase + k
            for i in range(nch):
                pv = pcol[row, pl.ds(i * LANES, LANES)]
                plsc.addupdate_scatter(accs[i], [idx], wpk * pv)
                zw = zcol[row, pl.ds(i * LANES, LANES)]
                plsc.addupdate_scatter(accs[nch + i], [idx], wzk * zw)
        return _

    lax.fori_loop(0, GROUPS, _group, 0)

    csz = NBUCKETS * LANES
    for j in range(nch):
        pltpu.sync_copy(accs[j],
                        outa_hbm.at[pl.ds((wid * nch + j) * csz, csz)])
        pltpu.sync_copy(accs[nch + j],
                        outb_hbm.at[pl.ds((wid * nch + j) * csz, csz)])


_seg_call = functools.partial(
    pl.kernel,
    out_type=(
        jax.ShapeDtypeStruct((NW * NBUCKETS * DSLICE,), jnp.float32),
        jax.ShapeDtypeStruct((NW * NBUCKETS * DSLICE,), jnp.float32),
    ),
    mesh=plsc.VectorSubcoreMesh(core_axis_name="c", subcore_axis_name="s"),
    compiler_params=pltpu.CompilerParams(needs_layout_passes=False,
                                         use_tc_tiling_on_sc=True),
    scratch_types=[
        pltpu.VMEM((HROWS, DSLICE), jnp.float32),
        pltpu.VMEM((HROWS, DSLICE), jnp.float32),
        pltpu.VMEM((HROWS,), jnp.int32),
        pltpu.VMEM((HROWS,), jnp.float32),
        pltpu.VMEM((HROWS,), jnp.float32),
    ] + [pltpu.VMEM((NBUCKETS * LANES,), jnp.float32)
         for _ in range(2 * (DSLICE // LANES))],

)(_seg_body)


NROWS_K3 = NQ * (DSLICE // LANES) * NBUCKETS


def _loss_body(a_ref, b_ref, d_ref, ta_ref, tb_ref, o_ref):
    a = a_ref[...]
    b = b_ref[...]
    at = a[:NROWS_K3] + a[NROWS_K3:]
    bt = b[:NROWS_K3] + b[NROWS_K3:]
    prod = at * bt
    v = jnp.sum(prod, axis=1, keepdims=True)
    ridx = lax.broadcasted_iota(jnp.int32, (NROWS_K3, 1), 0)
    sidx = lax.rem(ridx, NBUCKETS) // N_CLASSES
    d = d_ref[...]
    didx = lax.broadcasted_iota(jnp.int32, (BATCH, 1), 0) // (BATCH // N_SUBSETS)
    total = jnp.float32(0.0)
    for s in range(N_SUBSETS):
        ssum = jnp.sum(jnp.where(sidx == s, v, 0.0))
        dsum = jnp.sum(jnp.where(didx == s, d, 0.0))
        eq = ta_ref[:, s:s + 1] == tb_ref[s:s + 1, :]
        npairs = 0.5 * (jnp.sum(eq.astype(jnp.float32))
                        - jnp.float32(BATCH // N_SUBSETS))
        npairs = jnp.maximum(npairs, 1.0)
        total = total + jnp.float32(-0.5) * (ssum - dsum) / npairs
    o_ref[...] = jnp.full((1, 1), total / N_SUBSETS, jnp.float32)


def kernel(ps, zs, extra, targets):
    del extra
    wp, wz, d = pl.pallas_call(
        _norm_body,
        out_shape=(
            jax.ShapeDtypeStruct((BATCH, 1), jnp.float32),
            jax.ShapeDtypeStruct((BATCH, 1), jnp.float32),
            jax.ShapeDtypeStruct((BATCH, 1), jnp.float32),
        ),
    )(ps, zs)
    outa, outb = _seg_call(ps, zs, targets, wp.reshape(BATCH),
                           wz.reshape(BATCH))
    tb = targets.reshape(N_SUBSETS, BATCH // N_SUBSETS)
    ta = tb.T
    loss = pl.pallas_call(
        _loss_body,
        out_shape=jax.ShapeDtypeStruct((1, 1), jnp.float32),
    )(outa.reshape(2 * NROWS_K3, LANES), outb.reshape(2 * NROWS_K3, LANES),
      d, ta, tb)
    return loss[0, 0]


# speedup vs baseline: 1.2668x; 1.2668x over previous
import functools

import jax
import jax.numpy as jnp
from jax import lax
from jax.experimental import pallas as pl
from jax.experimental.pallas import tpu as pltpu
from jax.experimental.pallas import tpu_sc as plsc

N_SUBSETS = 3
BATCH = 768
DIM = 2048
N_CLASSES = 16
NBUCKETS = N_SUBSETS * N_CLASSES
LANES = 16
NW = 32
NQ = 16
NH = 2
DSLICE = DIM // NQ
HROWS = BATCH // NH
SUBSET_SHIFT = 8
GROUPS = HROWS // LANES


def _norm_body(p_ref, z_ref, wp_ref, wz_ref, d_ref):
    p = p_ref[...]
    z = z_ref[...]
    pp = jnp.sum(p * p, axis=1, keepdims=True)
    zz = jnp.sum(z * z, axis=1, keepdims=True)
    pz = jnp.sum(p * z, axis=1, keepdims=True)
    wp = 1.0 / jnp.maximum(jnp.sqrt(pp), 1e-8)
    wz = 1.0 / jnp.maximum(jnp.sqrt(zz), 1e-8)
    wp_ref[...] = wp
    wz_ref[...] = wz
    d_ref[...] = wp * wz * pz


def _seg_body(ps_hbm, zs_hbm, tgt_hbm, wp_hbm, wz_hbm,
              outa_hbm, outb_hbm, pcol, zcol, tbuf, wpbuf, wzbuf,
              acc_a, acc_b, sem_p, sem_z):
    cid = lax.axis_index("c")
    sid = lax.axis_index("s")
    wid = sid * 2 + cid
    h = wid // NQ
    q = wid % NQ
    rbase = h * HROWS
    nch = DSLICE // LANES

    cp = pltpu.async_copy(
        ps_hbm.at[pl.ds(rbase, HROWS), pl.ds(q * DSLICE, DSLICE)], pcol,
        sem_p)
    cz = pltpu.async_copy(
        zs_hbm.at[pl.ds(rbase, HROWS), pl.ds(q * DSLICE, DSLICE)], zcol,
        sem_z)
    pltpu.sync_copy(tgt_hbm.at[pl.ds(rbase, HROWS)], tbuf)
    pltpu.sync_copy(wp_hbm.at[pl.ds(rbase, HROWS)], wpbuf)
    pltpu.sync_copy(wz_hbm.at[pl.ds(rbase, HROWS)], wzbuf)

    zv = jnp.zeros((LANES,), jnp.float32)

    def _zero(i, c):
        acc_a[pl.ds(i * LANES, LANES)] = zv
        acc_b[pl.ds(i * LANES, LANES)] = zv
        return c

    lax.fori_loop(0, NBUCKETS * DSLICE // LANES, _zero, 0)
    cp.wait()
    cz.wait()

    iota = lax.iota(jnp.int32, LANES)
    gdn = lax.GatherDimensionNumbers(
        offset_dims=(), collapsed_slice_dims=(0,), start_index_map=(0,))

    def _group(g, _):
        gbase = g * LANES
        tv = tbuf[pl.ds(gbase, LANES)]
        wpv = wpbuf[pl.ds(gbase, LANES)]
        wzv = wzbuf[pl.ds(gbase, LANES)]
        rows = iota + jnp.full((LANES,), rbase + gbase, jnp.int32)
        offv = (lax.shift_right_arithmetic(rows, SUBSET_SHIFT) * N_CLASSES
                + tv) * DSLICE
        for k in range(LANES):
            kfull = jnp.full((LANES, 1), k, jnp.int32)
            bok = lax.gather(offv, kfull, gdn, (1,),
                             mode=lax.GatherScatterMode.PROMISE_IN_BOUNDS)
            wpk = jnp.full((LANES,), wpv[k], jnp.float32)
            wzk = jnp.full((LANES,), wzv[k], jnp.float32)
            row = gbase + k
            for i in range(nch):
                idx = bok + (iota + jnp.full((LANES,), i * LANES, jnp.int32))
                pv = pcol[row, pl.ds(i * LANES, LANES)]
                plsc.addupdate_scatter(acc_a, [idx], wpk * pv)
                zw = zcol[row, pl.ds(i * LANES, LANES)]
                plsc.addupdate_scatter(acc_b, [idx], wzk * zw)
        return _

    lax.fori_loop(0, GROUPS, _group, 0)

    asz = NBUCKETS * DSLICE
    pltpu.sync_copy(acc_a, outa_hbm.at[pl.ds(wid * asz, asz)])
    pltpu.sync_copy(acc_b, outb_hbm.at[pl.ds(wid * asz, asz)])


_seg_call = functools.partial(
    pl.kernel,
    out_type=(
        jax.ShapeDtypeStruct((NW * NBUCKETS * DSLICE,), jnp.float32),
        jax.ShapeDtypeStruct((NW * NBUCKETS * DSLICE,), jnp.float32),
    ),
    mesh=plsc.VectorSubcoreMesh(core_axis_name="c", subcore_axis_name="s"),
    compiler_params=pltpu.CompilerParams(needs_layout_passes=False,
                                         use_tc_tiling_on_sc=True),
    scratch_types=[
        pltpu.VMEM((HROWS, DSLICE), jnp.float32),
        pltpu.VMEM((HROWS, DSLICE), jnp.float32),
        pltpu.VMEM((HROWS,), jnp.int32),
        pltpu.VMEM((HROWS,), jnp.float32),
        pltpu.VMEM((HROWS,), jnp.float32),
        pltpu.VMEM((NBUCKETS * DSLICE,), jnp.float32),
        pltpu.VMEM((NBUCKETS * DSLICE,), jnp.float32),
        pltpu.SemaphoreType.DMA,
        pltpu.SemaphoreType.DMA,
    ],
)(_seg_body)


NROWS_K3 = NQ * NBUCKETS


def _loss_body(a_ref, b_ref, d_ref, ta_ref, tb_ref, o_ref):
    a = a_ref[...]
    b = b_ref[...]
    at = a[:NROWS_K3] + a[NROWS_K3:]
    bt = b[:NROWS_K3] + b[NROWS_K3:]
    prod = at * bt
    v = jnp.sum(prod, axis=1, keepdims=True)
    ridx = lax.broadcasted_iota(jnp.int32, (NROWS_K3, 1), 0)
    sidx = lax.rem(ridx, NBUCKETS) // N_CLASSES
    d = d_ref[...]
    didx = lax.broadcasted_iota(jnp.int32, (BATCH, 1), 0) // (BATCH // N_SUBSETS)
    total = jnp.float32(0.0)
    for s in range(N_SUBSETS):
        ssum = jnp.sum(jnp.where(sidx == s, v, 0.0))
        dsum = jnp.sum(jnp.where(didx == s, d, 0.0))
        eq = ta_ref[:, s:s + 1] == tb_ref[s:s + 1, :]
        npairs = 0.5 * (jnp.sum(eq.astype(jnp.float32))
                        - jnp.float32(BATCH // N_SUBSETS))
        npairs = jnp.maximum(npairs, 1.0)
        total = total + jnp.float32(-0.5) * (ssum - dsum) / npairs
    o_ref[...] = jnp.full((1, 1), total / N_SUBSETS, jnp.float32)


def kernel(ps, zs, extra, targets):
    del extra
    wp, wz, d = pl.pallas_call(
        _norm_body,
        out_shape=(
            jax.ShapeDtypeStruct((BATCH, 1), jnp.float32),
            jax.ShapeDtypeStruct((BATCH, 1), jnp.float32),
            jax.ShapeDtypeStruct((BATCH, 1), jnp.float32),
        ),
    )(ps, zs)
    outa, outb = _seg_call(ps, zs, targets, wp.reshape(BATCH),
                           wz.reshape(BATCH))
    tb = targets.reshape(N_SUBSETS, BATCH // N_SUBSETS)
    ta = tb.T
    loss = pl.pallas_call(
        _loss_body,
        out_shape=jax.ShapeDtypeStruct((1, 1), jnp.float32),
    )(outa.reshape(2 * NROWS_K3, DSLICE), outb.reshape(2 * NROWS_K3, DSLICE),
      d, ta, tb)
    return loss[0, 0]
